# Initial kernel scaffold; baseline (speedup 1.0000x reference)
import jax
import jax.numpy as jnp
from jax.experimental import pallas as pl


def kernel(x, edge_index, batch, ee_w1, ee_b1, ee_w2, ee_b2, ne_w1a, ne_b1a, ne_w1b, ne_b1b, ne_w2a, ne_b2a, ne_w2b, ne_b2b, ge_w1, ge_b1, ge_w2, ge_b2, ed_w1, ed_b1, ed_w2, ed_b2, nd_w1a, nd_b1a, nd_w1b, nd_b1b, nd_w2a, nd_b2a, nd_w2b, nd_b2b):
    raise NotImplementedError("write your pallas kernel here")


# speedup vs baseline: 2.4336x; 2.4336x over previous
import functools
import jax
import jax.numpy as jnp
from jax import lax
from jax.experimental import pallas as pl
from jax.experimental.pallas import tpu as pltpu
from jax.experimental.pallas import tpu_sc as plsc

N = 50000
E = 1600000
G = 64
NC = 2
NS = 16
NW = NC * NS
CH = 80
SUB = 8
CE = SUB * CH
Q = E // CE
KG = 6
RG = Q // (NW * KG)
KS = 1
RS = Q // (NW * KS)
KN = 3
RN = Q // (NW * KN)
QTAIL = 4
TPS = N // NS

def _mesh():
    return plsc.VectorSubcoreMesh(core_axis_name="c", subcore_axis_name="s")


def _hi(p):
    return jnp.dot(p[0], p[1], precision=lax.Precision.HIGHEST)


def _dot(a, b):
    return jnp.dot(a, b, precision=lax.Precision.HIGHEST)


def _gather_body(tab, row3, col3, gxr, gxc, idx_r, idx_c, rows_r, rows_c, sem):
    wid = lax.axis_index("s") * NC + lax.axis_index("c")

    def do_chunks(a, nk):
        pltpu.sync_copy(row3.at[pl.ds(a, nk)], idx_r.at[pl.ds(0, nk)])
        pltpu.sync_copy(col3.at[pl.ds(a, nk)], idx_c.at[pl.ds(0, nk)])
        hs = []
        for k in range(nk):
            for j in range(SUB):
                hs.append(pltpu.async_copy(
                    tab.at[idx_r.at[k, j]], rows_r.at[k, j], sem))
                hs.append(pltpu.async_copy(
                    tab.at[idx_c.at[k, j]], rows_c.at[k, j], sem))
        for h in hs:
            h.wait()
        pltpu.sync_copy(rows_r.at[pl.ds(0, nk)], gxr.at[pl.ds(a, nk)])
        pltpu.sync_copy(rows_c.at[pl.ds(0, nk)], gxc.at[pl.ds(a, nk)])

    def round_(r, carry):
        do_chunks((r * NW + wid) * KG, KG)
        return carry

    lax.fori_loop(0, RG, round_, 0)

    @pl.when(wid < QTAIL)
    def _():
        do_chunks(RG * NW * KG + wid, 1)


def _sc_gather(tab, row3, col3):
    f = functools.partial(
        pl.kernel,
        mesh=_mesh(),
        compiler_params=pltpu.CompilerParams(use_tc_tiling_on_sc=False),
        out_type=(
            jax.ShapeDtypeStruct((Q, SUB, CH, 8), jnp.float32),
            jax.ShapeDtypeStruct((Q, SUB, CH, 8), jnp.float32),
        ),
        scratch_types=[
            pltpu.VMEM((KG, SUB, CH), jnp.int32),
            pltpu.VMEM((KG, SUB, CH), jnp.int32),
            pltpu.VMEM((KG, SUB, CH, 8), jnp.float32),
            pltpu.VMEM((KG, SUB, CH, 8), jnp.float32),
            pltpu.SemaphoreType.DMA,
        ],
    )(_gather_body)
    return f(tab, row3, col3)


def _scatter32(h4, col3, zer):
    def body(h4, col3, zer, acc_out, acc_sh, idx_v, rows_v, sem):
        c = lax.axis_index("c")
        s = lax.axis_index("s")
        wid = s * NC + c
        pltpu.sync_copy(zer, acc_sh.at[pl.ds(s * TPS, TPS)])
        plsc.subcore_barrier()

        def do_chunks(a, nk):
            pltpu.sync_copy(col3.at[pl.ds(a, nk)], idx_v.at[pl.ds(0, nk)])
            pltpu.sync_copy(h4.at[pl.ds(a, nk)], rows_v.at[pl.ds(0, nk)])
            hs = []
            for k in range(nk):
                for j in range(SUB):
                    hs.append(
                        pltpu.async_copy(
                            rows_v.at[k, j], acc_sh.at[idx_v.at[k, j]],
                            sem, add=True
                        )
                    )
            for h in hs:
                h.wait()

        def round_(r, carry):
            do_chunks((r * NW + wid) * KS, KS)
            return carry

        lax.fori_loop(0, RS, round_, 0)

        @pl.when(wid < QTAIL)
        def _():
            do_chunks(RS * NW * KS + wid, 1)

        plsc.subcore_barrier()
        pltpu.sync_copy(
            acc_sh.at[pl.ds(s * TPS, TPS)], acc_out.at[c, pl.ds(s * TPS, TPS)]
        )

    f = functools.partial(
        pl.kernel,
        mesh=_mesh(),
        compiler_params=pltpu.CompilerParams(use_tc_tiling_on_sc=False),
        out_type=jax.ShapeDtypeStruct((NC, N, 32), jnp.float32),
        scratch_types=[
            pltpu.VMEM_SHARED((N, 32), jnp.float32),
            pltpu.VMEM((KS, SUB, CH), jnp.int32),
            pltpu.VMEM((KS, SUB, CH, 32), jnp.float32),
            pltpu.SemaphoreType.DMA,
        ],
    )(body)
    return f(h4, col3, zer)


def _sc_count(col3, zer, ones):
    def body(col3, zer, ones, acc_out, acc_sh, idx_v, ones_v, sem):
        c = lax.axis_index("c")
        s = lax.axis_index("s")
        wid = s * NC + c
        pltpu.sync_copy(zer, acc_sh.at[pl.ds(s * TPS, TPS)])
        pltpu.sync_copy(ones, ones_v)
        plsc.subcore_barrier()

        def do_chunks(a, nk):
            pltpu.sync_copy(col3.at[pl.ds(a, nk)], idx_v.at[pl.ds(0, nk)])
            hs = []
            for k in range(nk):
                for j in range(SUB):
                    hs.append(
                        pltpu.async_copy(
                            ones_v, acc_sh.at[idx_v.at[k, j]], sem, add=True
                        )
                    )
            for h in hs:
                h.wait()

        def round_(r, carry):
            do_chunks((r * NW + wid) * KN, KN)
            return carry

        lax.fori_loop(0, RN, round_, 0)

        @pl.when(wid < QTAIL)
        def _():
            do_chunks(RN * NW * KN + wid, 1)

        plsc.subcore_barrier()
        pltpu.sync_copy(
            acc_sh.at[pl.ds(s * TPS, TPS)], acc_out.at[c, pl.ds(s * TPS, TPS)]
        )

    f = functools.partial(
        pl.kernel,
        mesh=_mesh(),
        compiler_params=pltpu.CompilerParams(use_tc_tiling_on_sc=False),
        out_type=jax.ShapeDtypeStruct((NC, N, 8), jnp.float32),
        scratch_types=[
            pltpu.VMEM_SHARED((N, 8), jnp.float32),
            pltpu.VMEM((KN, SUB, CH), jnp.int32),
            pltpu.VMEM((CH, 8), jnp.float32),
            pltpu.SemaphoreType.DMA,
        ],
    )(body)
    return f(col3, zer, ones)


def _enc_edge_body(xr, xc, W_er, W_ec, b_ee, Wna_x, Wna_e, b_na, ee_w2T,
                   ee_b2r, W_nb, b_nb, out):
    M1 = _dot(ee_w2T[...], Wna_e[...])
    c1 = b_na[...] + _dot(ee_b2r[...], Wna_e[...])
    a = xr[...][:, 0:4]
    b = xc[...][:, 0:4]
    ee_h1 = _dot(a, W_er[...]) + _dot(b, W_ec[...]) + b_ee[...]
    t = jnp.maximum(ee_h1, 0.0)
    ne_h1 = _dot(t, M1) + _dot(a, Wna_x[...]) + c1
    s = jnp.maximum(ne_h1, 0.0)
    out[...] = _dot(s, W_nb[...]) + b_nb[...]


def _dec_edge_body(gr, gc, Urow, Wd_r, Wd_c, b_d1, ed_w2T, ed_b2r, Wnd_x,
                   Wnd_eT, b_nd1a, W_ndb, b_ndb, out):
    M2 = _dot(ed_w2T[...], Wnd_eT[...])
    c2 = b_nd1a[...] + _dot(ed_b2r[...], Wnd_eT[...])
    a = gr[...]
    x1r = a[:, 0:2]
    x1c = gc[...][:, 0:2]
    gbits = a[:, 2:3].astype(jnp.int32)
    blk = a.shape[0]
    iota = lax.broadcasted_iota(jnp.int32, (blk, G), 1)
    onehot = (gbits == iota).astype(jnp.float32)
    ucontrib = _dot(onehot, Urow[...])
    ed_h1 = _dot(x1r, Wd_r[...]) + _dot(x1c, Wd_c[...]) + ucontrib + b_d1[...]
    t = jnp.maximum(ed_h1, 0.0)
    nd_h1 = _dot(t, M2) + _dot(x1r, Wnd_x[...]) + c2
    s = jnp.maximum(nd_h1, 0.0)
    out[...] = _dot(s, W_ndb[...]) + b_ndb[...]


def _node_enc_body(x, hsA, hsB, cA, cB, batch2, Wa_x, Wa_g, b_2a, W_2b, b_2b,
                   ge_w1T, ge_b1r, ge_w2T, ge_b2r, Wed_u,
                   x1_out, xd_out, cnt_out, gsum_out, gcnt_out, u_out,
                   urow_out):
    i = pl.program_id(0)
    ssum = hsA[...] + hsB[...]
    cnt = cA[...] + cB[...]
    agg = ssum / jnp.maximum(cnt, 1.0)
    x1 = _dot(
        jnp.maximum(_dot(x[...], Wa_x[...]) + _dot(agg, Wa_g[...]) + b_2a[...],
                    0.0),
        W_2b[...],
    ) + b_2b[...]
    x1_out[...] = x1
    cnt_out[...] = cnt
    bt = batch2[...]
    blk = bt.shape[0]
    xd_out[...] = jnp.concatenate(
        [x1, bt.astype(jnp.float32), jnp.zeros((blk, 5), jnp.float32)],
        axis=1,
    )
    iota = lax.broadcasted_iota(jnp.int32, (blk, G), 1)
    onehot = (bt == iota).astype(jnp.float32)
    part = lax.dot_general(onehot, x1, (((0,), (0,)), ((), ())),
                           precision=lax.Precision.HIGHEST)
    pcnt = lax.dot_general(onehot, jnp.ones((blk, 1), jnp.float32),
                           (((0,), (0,)), ((), ())),
                           precision=lax.Precision.HIGHEST)

    @pl.when(i == 0)
    def _():
        gsum_out[...] = jnp.zeros_like(gsum_out)
        gcnt_out[...] = jnp.zeros_like(gcnt_out)

    gsum_out[...] += part
    gcnt_out[...] += pcnt

    @pl.when(i == pl.num_programs(0) - 1)
    def _():
        xb = gsum_out[...] / jnp.maximum(gcnt_out[...], 1.0)
        u = _dot(
            jnp.maximum(_dot(xb, ge_w1T[...]) + ge_b1r[...], 0.0), ge_w2T[...]
        ) + ge_b2r[...]
        u_out[...] = u
        urow_out[...] = _dot(u, Wed_u[...])


def _node_dec_body(x1, hdA, hdB, cnt, batch2, u, Wo_x, Wo_a, Wo_u, b_o1,
                   Wo_b, b_o2, out):
    aggd = (hdA[...] + hdB[...]) / jnp.maximum(cnt[...], 1.0)
    bt = batch2[...]
    blk = bt.shape[0]
    iota = lax.broadcasted_iota(jnp.int32, (blk, G), 1)
    onehot = (bt == iota).astype(jnp.float32)
    ub = _dot(onehot, u[...])
    a = x1[...]
    h = jnp.maximum(
        _dot(a, Wo_x[...]) + _dot(aggd, Wo_a[...]) + _dot(ub, Wo_u[...])
        + b_o1[...],
        0.0,
    )
    out[...] = _dot(h, Wo_b[...]) + b_o2[...]


EB = 8000
NB = 2000


def _full(shape):
    return pl.BlockSpec(shape, lambda i: (0,) * len(shape))


def _blk(shape):
    return pl.BlockSpec(shape, lambda i: (i,) + (0,) * (len(shape) - 1))


def kernel(x, edge_index, batch,
           ee_w1, ee_b1, ee_w2, ee_b2,
           ne_w1a, ne_b1a, ne_w1b, ne_b1b, ne_w2a, ne_b2a, ne_w2b, ne_b2b,
           ge_w1, ge_b1, ge_w2, ge_b2,
           ed_w1, ed_b1, ed_w2, ed_b2,
           nd_w1a, nd_b1a, nd_w1b, nd_b1b, nd_w2a, nd_b2a, nd_w2b, nd_b2b):
    row3 = edge_index[0].reshape(Q, SUB, CH)
    col3 = edge_index[1].reshape(Q, SUB, CH)
    batch2 = batch.reshape(N, 1)
    zer32 = jnp.zeros((TPS, 32), jnp.float32)
    zer8 = jnp.zeros((TPS, 8), jnp.float32)
    ones8 = jnp.ones((CH, 8), jnp.float32)

    cacc = _sc_count(col3, zer8, ones8)
    cnt0 = cacc[0, :, 0:1]
    cnt1 = cacc[1, :, 0:1]

    x8 = jnp.concatenate([x, jnp.zeros((N, 4), jnp.float32)], axis=1)
    gxr, gxc = _sc_gather(x8, row3, col3)
    gxr = gxr.reshape(E, 8)
    gxc = gxc.reshape(E, 8)

    r1 = lambda v: v.reshape(1, -1)
    h3 = pl.pallas_call(
        _enc_edge_body,
        grid=(E // EB,),
        in_specs=[
            _blk((EB, 8)), _blk((EB, 8)),
            _full((4, 32)), _full((4, 32)), _full((1, 32)),
            _full((4, 32)), _full((32, 32)), _full((1, 32)),
            _full((32, 32)), _full((1, 32)), _full((32, 32)), _full((1, 32)),
        ],
        out_specs=_blk((EB, 32)),
        out_shape=jax.ShapeDtypeStruct((E, 32), jnp.float32),
    )(gxr, gxc,
      ee_w1[:, 0:4].T, ee_w1[:, 4:8].T, r1(ee_b1),
      ne_w1a[:, 0:4].T, ne_w1a[:, 4:36].T, r1(ne_b1a),
      ee_w2.T, r1(ee_b2), ne_w1b.T, r1(ne_b1b))

    accs = _scatter32(h3.reshape(Q, SUB, CH, 32), col3, zer32)

    x1, xd, cntN, gsum, gcnt, u, urow = pl.pallas_call(
        _node_enc_body,
        grid=(N // NB,),
        in_specs=[
            _blk((NB, 4)), _blk((NB, 32)), _blk((NB, 32)), _blk((NB, 1)),
            _blk((NB, 1)), _blk((NB, 1)),
            _full((4, 32)), _full((32, 32)), _full((1, 32)),
            _full((32, 2)), _full((1, 2)),
            _full((2, 32)), _full((1, 32)), _full((32, 32)), _full((1, 32)),
            _full((32, 32)),
        ],
        out_specs=[
            _blk((NB, 2)), _blk((NB, 8)), _blk((NB, 1)),
            _full((G, 2)), _full((G, 1)), _full((G, 32)), _full((G, 32)),
        ],
        out_shape=[
            jax.ShapeDtypeStruct((N, 2), jnp.float32),
            jax.ShapeDtypeStruct((N, 8), jnp.float32),
            jax.ShapeDtypeStruct((N, 1), jnp.float32),
            jax.ShapeDtypeStruct((G, 2), jnp.float32),
            jax.ShapeDtypeStruct((G, 1), jnp.float32),
            jax.ShapeDtypeStruct((G, 32), jnp.float32),
            jax.ShapeDtypeStruct((G, 32), jnp.float32),
        ],
    )(x, accs[0], accs[1], cnt0, cnt1, batch2,
      ne_w2a[:, 0:4].T, ne_w2a[:, 4:36].T, r1(ne_b2a),
      ne_w2b.T, r1(ne_b2b),
      ge_w1.T, r1(ge_b1), ge_w2.T, r1(ge_b2),
      ed_w1[:, 4:36].T)

    gdr, gdc = _sc_gather(xd, row3, col3)
    gdr = gdr.reshape(E, 8)
    gdc = gdc.reshape(E, 8)

    hd3 = pl.pallas_call(
        _dec_edge_body,
        grid=(E // EB,),
        in_specs=[
            _blk((EB, 8)), _blk((EB, 8)), _full((G, 32)),
            _full((2, 32)), _full((2, 32)), _full((1, 32)),
            _full((32, 32)), _full((1, 32)),
            _full((2, 32)), _full((32, 32)), _full((1, 32)),
            _full((32, 32)), _full((1, 32)),
        ],
        out_specs=_blk((EB, 32)),
        out_shape=jax.ShapeDtypeStruct((E, 32), jnp.float32),
    )(gdr, gdc, urow,
      ed_w1[:, 0:2].T, ed_w1[:, 2:4].T, r1(ed_b1),
      ed_w2.T, r1(ed_b2),
      nd_w1a[:, 0:2].T, nd_w1a[:, 2:34].T, r1(nd_b1a),
      nd_w1b.T, r1(nd_b1b))

    daccs = _scatter32(hd3.reshape(Q, SUB, CH, 32), col3, zer32)

    out = pl.pallas_call(
        _node_dec_body,
        grid=(N // NB,),
        in_specs=[
            _blk((NB, 2)), _blk((NB, 32)), _blk((NB, 32)), _blk((NB, 1)),
            _blk((NB, 1)), _full((G, 32)),
            _full((2, 32)), _full((32, 32)), _full((32, 32)), _full((1, 32)),
            _full((32, 4)), _full((1, 4)),
        ],
        out_specs=_blk((NB, 4)),
        out_shape=jax.ShapeDtypeStruct((N, 4), jnp.float32),
    )(x1, daccs[0], daccs[1], cntN, batch2, u,
      nd_w2a[:, 0:2].T, nd_w2a[:, 2:34].T, nd_w2a[:, 34:66].T, r1(nd_b2a),
      nd_w2b.T, r1(nd_b2b))

    return out


# speedup vs baseline: 5.9593x; 2.4488x over previous
import functools
import jax
import jax.numpy as jnp
from jax import lax
from jax.experimental import pallas as pl
from jax.experimental.pallas import tpu as pltpu
from jax.experimental.pallas import tpu_sc as plsc

N = 50000
E = 1600000
G = 64
NC = 2
NS = 16
NW = NC * NS
CH = 80
SUB = 8
CE = SUB * CH
Q = E // CE
KG = 6
RG = Q // (NW * KG)
KS = 1
RS = Q // (NW * KS)
KN = 3
RN = Q // (NW * KN)
QTAIL = 4
TPS = N // NS

def _mesh():
    return plsc.VectorSubcoreMesh(core_axis_name="c", subcore_axis_name="s")


def _hi(p):
    return jnp.dot(p[0], p[1], precision=lax.Precision.HIGHEST)


def _dot(a, b):
    return jnp.dot(a, b, precision=lax.Precision.HIGHEST)


def _dotf(a, b):
    return jnp.dot(a, b, precision=lax.Precision.DEFAULT)


def _gather_body(tab, row3, col3, gxr, gxc, idx_r, idx_c, rows_r, rows_c, sem):
    wid = lax.axis_index("s") * NC + lax.axis_index("c")

    def do_chunks(a, nk):
        pltpu.sync_copy(row3.at[pl.ds(a, nk)], idx_r.at[pl.ds(0, nk)])
        pltpu.sync_copy(col3.at[pl.ds(a, nk)], idx_c.at[pl.ds(0, nk)])
        hs = []
        for k in range(nk):
            for j in range(SUB):
                hs.append(pltpu.async_copy(
                    tab.at[idx_r.at[k, j]], rows_r.at[k, j], sem))
                hs.append(pltpu.async_copy(
                    tab.at[idx_c.at[k, j]], rows_c.at[k, j], sem))
        for h in hs:
            h.wait()
        pltpu.sync_copy(rows_r.at[pl.ds(0, nk)], gxr.at[pl.ds(a, nk)])
        pltpu.sync_copy(rows_c.at[pl.ds(0, nk)], gxc.at[pl.ds(a, nk)])

    def round_(r, carry):
        do_chunks((r * NW + wid) * KG, KG)
        return carry

    lax.fori_loop(0, RG, round_, 0)

    @pl.when(wid < QTAIL)
    def _():
        do_chunks(RG * NW * KG + wid, 1)


def _sc_gather(tab, row3, col3):
    f = functools.partial(
        pl.kernel,
        mesh=_mesh(),
        compiler_params=pltpu.CompilerParams(use_tc_tiling_on_sc=False),
        out_type=(
            jax.ShapeDtypeStruct((Q, SUB, CH, 8), jnp.float32),
            jax.ShapeDtypeStruct((Q, SUB, CH, 8), jnp.float32),
        ),
        scratch_types=[
            pltpu.VMEM((KG, SUB, CH), jnp.int32),
            pltpu.VMEM((KG, SUB, CH), jnp.int32),
            pltpu.VMEM((KG, SUB, CH, 8), jnp.float32),
            pltpu.VMEM((KG, SUB, CH, 8), jnp.float32),
            pltpu.SemaphoreType.DMA,
        ],
    )(_gather_body)
    return f(tab, row3, col3)


def _scatter32(h4, col3, zer):
    def body(h4, col3, zer, acc_out, acc_sh, idx_v, rows_v, sem):
        c = lax.axis_index("c")
        s = lax.axis_index("s")
        wid = s * NC + c
        pltpu.sync_copy(zer, acc_sh.at[pl.ds(s * TPS, TPS)])
        plsc.subcore_barrier()

        def do_chunks(a, nk):
            pltpu.sync_copy(col3.at[pl.ds(a, nk)], idx_v.at[pl.ds(0, nk)])
            pltpu.sync_copy(h4.at[pl.ds(a, nk)], rows_v.at[pl.ds(0, nk)])
            hs = []
            for k in range(nk):
                for j in range(SUB):
                    hs.append(
                        pltpu.async_copy(
                            rows_v.at[k, j], acc_sh.at[idx_v.at[k, j]],
                            sem, add=True
                        )
                    )
            for h in hs:
                h.wait()

        def round_(r, carry):
            do_chunks((r * NW + wid) * KS, KS)
            return carry

        lax.fori_loop(0, RS, round_, 0)

        @pl.when(wid < QTAIL)
        def _():
            do_chunks(RS * NW * KS + wid, 1)

        plsc.subcore_barrier()
        pltpu.sync_copy(
            acc_sh.at[pl.ds(s * TPS, TPS)], acc_out.at[c, pl.ds(s * TPS, TPS)]
        )

    f = functools.partial(
        pl.kernel,
        mesh=_mesh(),
        compiler_params=pltpu.CompilerParams(use_tc_tiling_on_sc=False),
        out_type=jax.ShapeDtypeStruct((NC, N, 32), jnp.float32),
        scratch_types=[
            pltpu.VMEM_SHARED((N, 32), jnp.float32),
            pltpu.VMEM((KS, SUB, CH), jnp.int32),
            pltpu.VMEM((KS, SUB, CH, 32), jnp.float32),
            pltpu.SemaphoreType.DMA,
        ],
    )(body)
    return f(h4, col3, zer)


def _sc_count(col3, zer, ones):
    def body(col3, zer, ones, acc_out, acc_sh, idx_v, ones_v, sem):
        c = lax.axis_index("c")
        s = lax.axis_index("s")
        wid = s * NC + c
        pltpu.sync_copy(zer, acc_sh.at[pl.ds(s * TPS, TPS)])
        pltpu.sync_copy(ones, ones_v)
        plsc.subcore_barrier()

        def do_chunks(a, nk):
            pltpu.sync_copy(col3.at[pl.ds(a, nk)], idx_v.at[pl.ds(0, nk)])
            hs = []
            for k in range(nk):
                for j in range(SUB):
                    hs.append(
                        pltpu.async_copy(
                            ones_v, acc_sh.at[idx_v.at[k, j]], sem, add=True
                        )
                    )
            for h in hs:
                h.wait()

        def round_(r, carry):
            do_chunks((r * NW + wid) * KN, KN)
            return carry

        lax.fori_loop(0, RN, round_, 0)

        @pl.when(wid < QTAIL)
        def _():
            do_chunks(RN * NW * KN + wid, 1)

        plsc.subcore_barrier()
        pltpu.sync_copy(
            acc_sh.at[pl.ds(s * TPS, TPS)], acc_out.at[c, pl.ds(s * TPS, TPS)]
        )

    f = functools.partial(
        pl.kernel,
        mesh=_mesh(),
        compiler_params=pltpu.CompilerParams(use_tc_tiling_on_sc=False),
        out_type=jax.ShapeDtypeStruct((NC, N, 8), jnp.float32),
        scratch_types=[
            pltpu.VMEM_SHARED((N, 8), jnp.float32),
            pltpu.VMEM((KN, SUB, CH), jnp.int32),
            pltpu.VMEM((CH, 8), jnp.float32),
            pltpu.SemaphoreType.DMA,
        ],
    )(body)
    return f(col3, zer, ones)


def _enc_edge_body(xr, xc, W_er, W_ec, b_ee, Wna_x, Wna_e, b_na, ee_w2T,
                   ee_b2r, W_nb, b_nb, out):
    M1 = _dot(ee_w2T[...], Wna_e[...])
    c1 = b_na[...] + _dot(ee_b2r[...], Wna_e[...])
    a = xr[...][:, 0:4]
    b = xc[...][:, 0:4]
    ab = jnp.concatenate([a, b], axis=1)
    W1 = jnp.concatenate([W_er[...], W_ec[...]], axis=0)
    ee_h1 = _dotf(ab, W1) + b_ee[...]
    t = jnp.maximum(ee_h1, 0.0)
    ta = jnp.concatenate([t, a], axis=1)
    W2 = jnp.concatenate([M1, Wna_x[...]], axis=0)
    ne_h1 = _dotf(ta, W2) + c1
    s = jnp.maximum(ne_h1, 0.0)
    out[...] = _dotf(s, W_nb[...]) + b_nb[...]


def _dec_edge_body(gr, gc, Urow, Wd_r, Wd_c, b_d1, ed_w2T, ed_b2r, Wnd_x,
                   Wnd_eT, b_nd1a, W_ndb, b_ndb, out):
    M2 = _dot(ed_w2T[...], Wnd_eT[...])
    c2 = b_nd1a[...] + _dot(ed_b2r[...], Wnd_eT[...])
    a = gr[...]
    x1r = a[:, 0:2]
    x1c = gc[...][:, 0:2]
    gbits = a[:, 2:3].astype(jnp.int32)
    blk = a.shape[0]
    iota = lax.broadcasted_iota(jnp.int32, (blk, G), 1)
    onehot = (gbits == iota).astype(jnp.float32)
    rcu = jnp.concatenate([x1r, x1c, onehot], axis=1)
    W1 = jnp.concatenate([Wd_r[...], Wd_c[...], Urow[...]], axis=0)
    ed_h1 = _dotf(rcu, W1) + b_d1[...]
    t = jnp.maximum(ed_h1, 0.0)
    ta = jnp.concatenate([t, x1r], axis=1)
    W2 = jnp.concatenate([M2, Wnd_x[...]], axis=0)
    nd_h1 = _dotf(ta, W2) + c2
    s = jnp.maximum(nd_h1, 0.0)
    out[...] = _dotf(s, W_ndb[...]) + b_ndb[...]


def _node_enc_body(x, hsA, hsB, cA, cB, batch2, Wa_x, Wa_g, b_2a, W_2b, b_2b,
                   ge_w1T, ge_b1r, ge_w2T, ge_b2r, Wed_u,
                   x1_out, xd_out, cnt_out, gsum_out, gcnt_out, u_out,
                   urow_out):
    i = pl.program_id(0)
    ssum = hsA[...] + hsB[...]
    cnt = cA[...] + cB[...]
    agg = ssum / jnp.maximum(cnt, 1.0)
    x1 = _dot(
        jnp.maximum(_dot(x[...], Wa_x[...]) + _dot(agg, Wa_g[...]) + b_2a[...],
                    0.0),
        W_2b[...],
    ) + b_2b[...]
    x1_out[...] = x1
    cnt_out[...] = cnt
    bt = batch2[...]
    blk = bt.shape[0]
    xd_out[...] = jnp.concatenate(
        [x1, bt.astype(jnp.float32), jnp.zeros((blk, 5), jnp.float32)],
        axis=1,
    )
    iota = lax.broadcasted_iota(jnp.int32, (blk, G), 1)
    onehot = (bt == iota).astype(jnp.float32)
    part = lax.dot_general(onehot, x1, (((0,), (0,)), ((), ())),
                           precision=lax.Precision.HIGHEST)
    pcnt = lax.dot_general(onehot, jnp.ones((blk, 1), jnp.float32),
                           (((0,), (0,)), ((), ())),
                           precision=lax.Precision.HIGHEST)

    @pl.when(i == 0)
    def _():
        gsum_out[...] = jnp.zeros_like(gsum_out)
        gcnt_out[...] = jnp.zeros_like(gcnt_out)

    gsum_out[...] += part
    gcnt_out[...] += pcnt

    @pl.when(i == pl.num_programs(0) - 1)
    def _():
        xb = gsum_out[...] / jnp.maximum(gcnt_out[...], 1.0)
        u = _dot(
            jnp.maximum(_dot(xb, ge_w1T[...]) + ge_b1r[...], 0.0), ge_w2T[...]
        ) + ge_b2r[...]
        u_out[...] = u
        urow_out[...] = _dot(u, Wed_u[...])


def _node_dec_body(x1, hdA, hdB, cnt, batch2, u, Wo_x, Wo_a, Wo_u, b_o1,
                   Wo_b, b_o2, out):
    aggd = (hdA[...] + hdB[...]) / jnp.maximum(cnt[...], 1.0)
    bt = batch2[...]
    blk = bt.shape[0]
    iota = lax.broadcasted_iota(jnp.int32, (blk, G), 1)
    onehot = (bt == iota).astype(jnp.float32)
    ub = _dot(onehot, u[...])
    a = x1[...]
    h = jnp.maximum(
        _dot(a, Wo_x[...]) + _dot(aggd, Wo_a[...]) + _dot(ub, Wo_u[...])
        + b_o1[...],
        0.0,
    )
    out[...] = _dot(h, Wo_b[...]) + b_o2[...]


EB = 8000
NB = 2000


def _full(shape):
    return pl.BlockSpec(shape, lambda i: (0,) * len(shape))


def _blk(shape):
    return pl.BlockSpec(shape, lambda i: (i,) + (0,) * (len(shape) - 1))


def kernel(x, edge_index, batch,
           ee_w1, ee_b1, ee_w2, ee_b2,
           ne_w1a, ne_b1a, ne_w1b, ne_b1b, ne_w2a, ne_b2a, ne_w2b, ne_b2b,
           ge_w1, ge_b1, ge_w2, ge_b2,
           ed_w1, ed_b1, ed_w2, ed_b2,
           nd_w1a, nd_b1a, nd_w1b, nd_b1b, nd_w2a, nd_b2a, nd_w2b, nd_b2b):
    row3 = edge_index[0].reshape(Q, SUB, CH)
    col3 = edge_index[1].reshape(Q, SUB, CH)
    batch2 = batch.reshape(N, 1)
    zer32 = jnp.zeros((TPS, 32), jnp.float32)
    zer8 = jnp.zeros((TPS, 8), jnp.float32)
    ones8 = jnp.ones((CH, 8), jnp.float32)

    cacc = _sc_count(col3, zer8, ones8)
    cnt0 = cacc[0, :, 0:1]
    cnt1 = cacc[1, :, 0:1]

    x8 = jnp.concatenate([x, jnp.zeros((N, 4), jnp.float32)], axis=1)
    gxr, gxc = _sc_gather(x8, row3, col3)
    gxr = gxr.reshape(E, 8)
    gxc = gxc.reshape(E, 8)

    r1 = lambda v: v.reshape(1, -1)
    h3 = pl.pallas_call(
        _enc_edge_body,
        grid=(E // EB,),
        in_specs=[
            _blk((EB, 8)), _blk((EB, 8)),
            _full((4, 32)), _full((4, 32)), _full((1, 32)),
            _full((4, 32)), _full((32, 32)), _full((1, 32)),
            _full((32, 32)), _full((1, 32)), _full((32, 32)), _full((1, 32)),
        ],
        out_specs=_blk((EB, 32)),
        out_shape=jax.ShapeDtypeStruct((E, 32), jnp.float32),
    )(gxr, gxc,
      ee_w1[:, 0:4].T, ee_w1[:, 4:8].T, r1(ee_b1),
      ne_w1a[:, 0:4].T, ne_w1a[:, 4:36].T, r1(ne_b1a),
      ee_w2.T, r1(ee_b2), ne_w1b.T, r1(ne_b1b))

    accs = _scatter32(h3.reshape(Q, SUB, CH, 32), col3, zer32)

    x1, xd, cntN, gsum, gcnt, u, urow = pl.pallas_call(
        _node_enc_body,
        grid=(N // NB,),
        in_specs=[
            _blk((NB, 4)), _blk((NB, 32)), _blk((NB, 32)), _blk((NB, 1)),
            _blk((NB, 1)), _blk((NB, 1)),
            _full((4, 32)), _full((32, 32)), _full((1, 32)),
            _full((32, 2)), _full((1, 2)),
            _full((2, 32)), _full((1, 32)), _full((32, 32)), _full((1, 32)),
            _full((32, 32)),
        ],
        out_specs=[
            _blk((NB, 2)), _blk((NB, 8)), _blk((NB, 1)),
            _full((G, 2)), _full((G, 1)), _full((G, 32)), _full((G, 32)),
        ],
        out_shape=[
            jax.ShapeDtypeStruct((N, 2), jnp.float32),
            jax.ShapeDtypeStruct((N, 8), jnp.float32),
            jax.ShapeDtypeStruct((N, 1), jnp.float32),
            jax.ShapeDtypeStruct((G, 2), jnp.float32),
            jax.ShapeDtypeStruct((G, 1), jnp.float32),
            jax.ShapeDtypeStruct((G, 32), jnp.float32),
            jax.ShapeDtypeStruct((G, 32), jnp.float32),
        ],
    )(x, accs[0], accs[1], cnt0, cnt1, batch2,
      ne_w2a[:, 0:4].T, ne_w2a[:, 4:36].T, r1(ne_b2a),
      ne_w2b.T, r1(ne_b2b),
      ge_w1.T, r1(ge_b1), ge_w2.T, r1(ge_b2),
      ed_w1[:, 4:36].T)

    gdr, gdc = _sc_gather(xd, row3, col3)
    gdr = gdr.reshape(E, 8)
    gdc = gdc.reshape(E, 8)

    hd3 = pl.pallas_call(
        _dec_edge_body,
        grid=(E // EB,),
        in_specs=[
            _blk((EB, 8)), _blk((EB, 8)), _full((G, 32)),
            _full((2, 32)), _full((2, 32)), _full((1, 32)),
            _full((32, 32)), _full((1, 32)),
            _full((2, 32)), _full((32, 32)), _full((1, 32)),
            _full((32, 32)), _full((1, 32)),
        ],
        out_specs=_blk((EB, 32)),
        out_shape=jax.ShapeDtypeStruct((E, 32), jnp.float32),
    )(gdr, gdc, urow,
      ed_w1[:, 0:2].T, ed_w1[:, 2:4].T, r1(ed_b1),
      ed_w2.T, r1(ed_b2),
      nd_w1a[:, 0:2].T, nd_w1a[:, 2:34].T, r1(nd_b1a),
      nd_w1b.T, r1(nd_b1b))

    daccs = _scatter32(hd3.reshape(Q, SUB, CH, 32), col3, zer32)

    out = pl.pallas_call(
        _node_dec_body,
        grid=(N // NB,),
        in_specs=[
            _blk((NB, 2)), _blk((NB, 32)), _blk((NB, 32)), _blk((NB, 1)),
            _blk((NB, 1)), _full((G, 32)),
            _full((2, 32)), _full((32, 32)), _full((32, 32)), _full((1, 32)),
            _full((32, 4)), _full((1, 4)),
        ],
        out_specs=_blk((NB, 4)),
        out_shape=jax.ShapeDtypeStruct((N, 4), jnp.float32),
    )(x1, daccs[0], daccs[1], cntN, batch2, u,
      nd_w2a[:, 0:2].T, nd_w2a[:, 2:34].T, nd_w2a[:, 34:66].T, r1(nd_b2a),
      nd_w2b.T, r1(nd_b2b))

    return out


# speedup vs baseline: 12.9054x; 2.1656x over previous
import functools
import jax
import jax.numpy as jnp
from jax import lax
from jax.experimental import pallas as pl
from jax.experimental.pallas import tpu as pltpu
from jax.experimental.pallas import tpu_sc as plsc

N = 50000
E = 1600000
G = 64
NC = 2
NS = 16
NW = NC * NS
CH = 80
SUB = 8
CE = SUB * CH
Q = E // CE
KG = 6
RG = Q // (NW * KG)
KS = 1
RS = Q // (NW * KS)
KN = 3
RN = Q // (NW * KN)
QTAIL = 4
TPS = N // NS

def _mesh():
    return plsc.VectorSubcoreMesh(core_axis_name="c", subcore_axis_name="s")


def _hi(p):
    return jnp.dot(p[0], p[1], precision=lax.Precision.HIGHEST)


def _dot(a, b):
    return jnp.dot(a, b, precision=lax.Precision.HIGHEST)


def _dotf(a, b):
    ah = a.astype(jnp.bfloat16)
    al = (a - ah.astype(jnp.float32)).astype(jnp.bfloat16)
    bh = b.astype(jnp.bfloat16)
    bl = (b - bh.astype(jnp.float32)).astype(jnp.bfloat16)

    def d(p, q):
        return lax.dot_general(p, q, (((1,), (0,)), ((), ())),
                               preferred_element_type=jnp.float32)

    return d(ah, bh) + d(ah, bl) + d(al, bh)


def _gather_body(tab, row3, col3, gxr, gxc, idx_r, idx_c, rows_r, rows_c, sem):
    wid = lax.axis_index("s") * NC + lax.axis_index("c")

    def do_chunks(a, nk):
        pltpu.sync_copy(row3.at[pl.ds(a, nk)], idx_r.at[pl.ds(0, nk)])
        pltpu.sync_copy(col3.at[pl.ds(a, nk)], idx_c.at[pl.ds(0, nk)])
        hs = []
        for k in range(nk):
            for j in range(SUB):
                hs.append(pltpu.async_copy(
                    tab.at[idx_r.at[k, j]], rows_r.at[k, j], sem))
                hs.append(pltpu.async_copy(
                    tab.at[idx_c.at[k, j]], rows_c.at[k, j], sem))
        for h in hs:
            h.wait()
        pltpu.sync_copy(rows_r.at[pl.ds(0, nk)], gxr.at[pl.ds(a, nk)])
        pltpu.sync_copy(rows_c.at[pl.ds(0, nk)], gxc.at[pl.ds(a, nk)])

    def round_(r, carry):
        do_chunks((r * NW + wid) * KG, KG)
        return carry

    lax.fori_loop(0, RG, round_, 0)

    @pl.when(wid < QTAIL)
    def _():
        do_chunks(RG * NW * KG + wid, 1)


def _sc_gather(tab, row3, col3):
    f = functools.partial(
        pl.kernel,
        mesh=_mesh(),
        compiler_params=pltpu.CompilerParams(use_tc_tiling_on_sc=False),
        out_type=(
            jax.ShapeDtypeStruct((Q, SUB, CH, 8), jnp.float32),
            jax.ShapeDtypeStruct((Q, SUB, CH, 8), jnp.float32),
        ),
        scratch_types=[
            pltpu.VMEM((KG, SUB, CH), jnp.int32),
            pltpu.VMEM((KG, SUB, CH), jnp.int32),
            pltpu.VMEM((KG, SUB, CH, 8), jnp.float32),
            pltpu.VMEM((KG, SUB, CH, 8), jnp.float32),
            pltpu.SemaphoreType.DMA,
        ],
    )(_gather_body)
    return f(tab, row3, col3)


def _scatter32(h4, col3, zer):
    def body(h4, col3, zer, acc_out, acc_sh, idx_v, rows_v, sem):
        c = lax.axis_index("c")
        s = lax.axis_index("s")
        wid = s * NC + c
        pltpu.sync_copy(zer, acc_sh.at[pl.ds(s * TPS, TPS)])
        plsc.subcore_barrier()

        def do_chunks(a, nk):
            pltpu.sync_copy(col3.at[pl.ds(a, nk)], idx_v.at[pl.ds(0, nk)])
            pltpu.sync_copy(h4.at[pl.ds(a, nk)], rows_v.at[pl.ds(0, nk)])
            hs = []
            for k in range(nk):
                for j in range(SUB):
                    hs.append(
                        pltpu.async_copy(
                            rows_v.at[k, j], acc_sh.at[idx_v.at[k, j]],
                            sem, add=True
                        )
                    )
            for h in hs:
                h.wait()

        def round_(r, carry):
            do_chunks((r * NW + wid) * KS, KS)
            return carry

        lax.fori_loop(0, RS, round_, 0)

        @pl.when(wid < QTAIL)
        def _():
            do_chunks(RS * NW * KS + wid, 1)

        plsc.subcore_barrier()
        pltpu.sync_copy(
            acc_sh.at[pl.ds(s * TPS, TPS)], acc_out.at[c, pl.ds(s * TPS, TPS)]
        )

    f = functools.partial(
        pl.kernel,
        mesh=_mesh(),
        compiler_params=pltpu.CompilerParams(use_tc_tiling_on_sc=False),
        out_type=jax.ShapeDtypeStruct((NC, N, 32), jnp.float32),
        scratch_types=[
            pltpu.VMEM_SHARED((N, 32), jnp.float32),
            pltpu.VMEM((KS, SUB, CH), jnp.int32),
            pltpu.VMEM((KS, SUB, CH, 32), jnp.float32),
            pltpu.SemaphoreType.DMA,
        ],
    )(body)
    return f(h4, col3, zer)


def _sc_count(col3, zer, ones):
    def body(col3, zer, ones, acc_out, acc_sh, idx_v, ones_v, sem):
        c = lax.axis_index("c")
        s = lax.axis_index("s")
        wid = s * NC + c
        pltpu.sync_copy(zer, acc_sh.at[pl.ds(s * TPS, TPS)])
        pltpu.sync_copy(ones, ones_v)
        plsc.subcore_barrier()

        def do_chunks(a, nk):
            pltpu.sync_copy(col3.at[pl.ds(a, nk)], idx_v.at[pl.ds(0, nk)])
            hs = []
            for k in range(nk):
                for j in range(SUB):
                    hs.append(
                        pltpu.async_copy(
                            ones_v, acc_sh.at[idx_v.at[k, j]], sem, add=True
                        )
                    )
            for h in hs:
                h.wait()

        def round_(r, carry):
            do_chunks((r * NW + wid) * KN, KN)
            return carry

        lax.fori_loop(0, RN, round_, 0)

        @pl.when(wid < QTAIL)
        def _():
            do_chunks(RN * NW * KN + wid, 1)

        plsc.subcore_barrier()
        pltpu.sync_copy(
            acc_sh.at[pl.ds(s * TPS, TPS)], acc_out.at[c, pl.ds(s * TPS, TPS)]
        )

    f = functools.partial(
        pl.kernel,
        mesh=_mesh(),
        compiler_params=pltpu.CompilerParams(use_tc_tiling_on_sc=False),
        out_type=jax.ShapeDtypeStruct((NC, N, 8), jnp.float32),
        scratch_types=[
            pltpu.VMEM_SHARED((N, 8), jnp.float32),
            pltpu.VMEM((KN, SUB, CH), jnp.int32),
            pltpu.VMEM((CH, 8), jnp.float32),
            pltpu.SemaphoreType.DMA,
        ],
    )(body)
    return f(col3, zer, ones)


def _enc_edge_body(xr, xc, B16r, B16c, bee16, B4m1, B16nax, c14, B4nb, bnb4,
                   out):
    p = xr.shape[0]
    a = xr[...]
    t16 = jnp.maximum(
        _dotf(a, B16r[...]) + _dotf(xc[...], B16c[...]) + bee16[...],
        0.0)
    t4 = t16.reshape(p * 4, 128)
    xw4 = _dotf(a, B16nax[...]).reshape(p * 4, 128)
    ne4 = _dotf(t4, B4m1[...]) + xw4 + c14[...]
    s4 = jnp.maximum(ne4, 0.0)
    out[...] = _dotf(s4, B4nb[...]) + bnb4[...]


def _dec_edge_body(gr, gc, Urow, B16a, B16b, Sg16, i64, umask, bd14, B4m2,
                    B16ndx, c24, B4ndb, bndb4, out):
    p = gr.shape[0]
    a = gr[...]
    b = gc[...]
    gb = jnp.dot(a, Sg16[...],
                 preferred_element_type=jnp.float32).reshape(p * 4, 256)
    onehot = (gb == i64[...]).astype(jnp.float32)
    bdu = jnp.tile(Urow[...], (4, 4)) * umask[...]
    ucontrib = _dotf(onehot, bdu)
    rc4 = (_dotf(a, B16a[...]) + _dotf(b, B16b[...])).reshape(p * 4, 128)
    ed4 = rc4 + ucontrib + bd14[...]
    t = jnp.maximum(ed4, 0.0)
    ndx4 = _dotf(a, B16ndx[...]).reshape(p * 4, 128)
    nd4 = _dotf(t, B4m2[...]) + ndx4 + c24[...]
    s = jnp.maximum(nd4, 0.0)
    out[...] = _dotf(s, B4ndb[...]) + bndb4[...]


def _node_enc_body(x, hsA, hsB, cA, cB, batch2, Wa_x, Wa_g, b_2a, W_2b, b_2b,
                   ge_w1T, ge_b1r, ge_w2T, ge_b2r, Wed_u,
                   x1_out, xd_out, cnt_out, gsum_out, gcnt_out, u_out,
                   urow_out):
    i = pl.program_id(0)
    ssum = hsA[...] + hsB[...]
    cnt = cA[...] + cB[...]
    agg = ssum / jnp.maximum(cnt, 1.0)
    x1 = _dot(
        jnp.maximum(_dot(x[...], Wa_x[...]) + _dot(agg, Wa_g[...]) + b_2a[...],
                    0.0),
        W_2b[...],
    ) + b_2b[...]
    x1_out[...] = x1
    cnt_out[...] = cnt
    bt = batch2[...]
    blk = bt.shape[0]
    xd_out[...] = jnp.concatenate(
        [x1, bt.astype(jnp.float32), jnp.zeros((blk, 5), jnp.float32)],
        axis=1,
    )
    iota = lax.broadcasted_iota(jnp.int32, (blk, G), 1)
    onehot = (bt == iota).astype(jnp.float32)
    part = lax.dot_general(onehot, x1, (((0,), (0,)), ((), ())),
                           precision=lax.Precision.HIGHEST)
    pcnt = lax.dot_general(onehot, jnp.ones((blk, 1), jnp.float32),
                           (((0,), (0,)), ((), ())),
                           precision=lax.Precision.HIGHEST)

    @pl.when(i == 0)
    def _():
        gsum_out[...] = jnp.zeros_like(gsum_out)
        gcnt_out[...] = jnp.zeros_like(gcnt_out)

    gsum_out[...] += part
    gcnt_out[...] += pcnt

    @pl.when(i == pl.num_programs(0) - 1)
    def _():
        xb = gsum_out[...] / jnp.maximum(gcnt_out[...], 1.0)
        u = _dot(
            jnp.maximum(_dot(xb, ge_w1T[...]) + ge_b1r[...], 0.0), ge_w2T[...]
        ) + ge_b2r[...]
        u_out[...] = u
        urow_out[...] = _dot(u, Wed_u[...])


def _node_dec_body(x1, hdA, hdB, cnt, batch2, u, Wo_x, Wo_a, Wo_u, b_o1,
                   Wo_b, b_o2, out):
    aggd = (hdA[...] + hdB[...]) / jnp.maximum(cnt[...], 1.0)
    bt = batch2[...]
    blk = bt.shape[0]
    iota = lax.broadcasted_iota(jnp.int32, (blk, G), 1)
    onehot = (bt == iota).astype(jnp.float32)
    ub = _dot(onehot, u[...])
    a = x1[...]
    h = jnp.maximum(
        _dot(a, Wo_x[...]) + _dot(aggd, Wo_a[...]) + _dot(ub, Wo_u[...])
        + b_o1[...],
        0.0,
    )
    out[...] = _dot(h, Wo_b[...]) + b_o2[...]


EB = 12800
NB = 2000


def _full(shape):
    return pl.BlockSpec(shape, lambda i: (0,) * len(shape))


def _blk(shape):
    return pl.BlockSpec(shape, lambda i: (i,) + (0,) * (len(shape) - 1))


def kernel(x, edge_index, batch,
           ee_w1, ee_b1, ee_w2, ee_b2,
           ne_w1a, ne_b1a, ne_w1b, ne_b1b, ne_w2a, ne_b2a, ne_w2b, ne_b2b,
           ge_w1, ge_b1, ge_w2, ge_b2,
           ed_w1, ed_b1, ed_w2, ed_b2,
           nd_w1a, nd_b1a, nd_w1b, nd_b1b, nd_w2a, nd_b2a, nd_w2b, nd_b2b):
    row3 = edge_index[0].reshape(Q, SUB, CH)
    col3 = edge_index[1].reshape(Q, SUB, CH)
    batch2 = batch.reshape(N, 1)
    zer32 = jnp.zeros((TPS, 32), jnp.float32)
    zer8 = jnp.zeros((TPS, 8), jnp.float32)
    ones8 = jnp.ones((CH, 8), jnp.float32)

    cacc = _sc_count(col3, zer8, ones8)
    cnt0 = cacc[0, :, 0:1]
    cnt1 = cacc[1, :, 0:1]

    x8 = jnp.concatenate([x, jnp.zeros((N, 4), jnp.float32)], axis=1)
    gxr, gxc = _sc_gather(x8, row3, col3)
    xr_p = gxr.reshape(E * 8 // 128, 128)
    xc_p = gxc.reshape(E * 8 // 128, 128)

    r1 = lambda v: v.reshape(1, -1)
    I16 = jnp.eye(16, dtype=jnp.float32)
    I4 = jnp.eye(4, dtype=jnp.float32)
    kron = jnp.kron
    M1 = (ne_w1a[:, 4:36] @ ee_w2).T
    c1 = ne_b1a + ee_b2 @ ne_w1a[:, 4:36].T
    z832 = jnp.zeros((8, 32), jnp.float32)
    A_r = z832.at[0:4].set(ee_w1[:, 0:4].T)
    A_c = z832.at[0:4].set(ee_w1[:, 4:8].T)
    A_nax = z832.at[0:4].set(ne_w1a[:, 0:4].T)
    B16r = kron(I16, A_r)
    B16c = kron(I16, A_c)
    B4m1 = kron(I4, M1)
    B16nax = kron(I16, A_nax)
    B4nb = kron(I4, ne_w1b.T)
    bee16 = jnp.tile(ee_b1, 16).reshape(1, 512)
    c14 = jnp.tile(c1, 4).reshape(1, 128)
    bnb4 = jnp.tile(ne_b1b, 4).reshape(1, 128)

    PB = EB // 16
    h128 = pl.pallas_call(
        _enc_edge_body,
        grid=(E // EB,),
        in_specs=[
            _blk((PB, 128)), _blk((PB, 128)),
            _full((128, 512)), _full((128, 512)), _full((1, 512)),
            _full((128, 128)), _full((128, 512)), _full((1, 128)),
            _full((128, 128)), _full((1, 128)),
        ],
        out_specs=_blk((EB // 4, 128)),
        out_shape=jax.ShapeDtypeStruct((E * 32 // 128, 128), jnp.float32),
    )(xr_p, xc_p, B16r, B16c, bee16, B4m1, B16nax, c14, B4nb, bnb4)

    accs = _scatter32(h128.reshape(Q, SUB, CH, 32), col3, zer32)

    x1, xd, cntN, gsum, gcnt, u, urow = pl.pallas_call(
        _node_enc_body,
        grid=(N // NB,),
        in_specs=[
            _blk((NB, 4)), _blk((NB, 32)), _blk((NB, 32)), _blk((NB, 1)),
            _blk((NB, 1)), _blk((NB, 1)),
            _full((4, 32)), _full((32, 32)), _full((1, 32)),
            _full((32, 2)), _full((1, 2)),
            _full((2, 32)), _full((1, 32)), _full((32, 32)), _full((1, 32)),
            _full((32, 32)),
        ],
        out_specs=[
            _blk((NB, 2)), _blk((NB, 8)), _blk((NB, 1)),
            _full((G, 2)), _full((G, 1)), _full((G, 32)), _full((G, 32)),
        ],
        out_shape=[
            jax.ShapeDtypeStruct((N, 2), jnp.float32),
            jax.ShapeDtypeStruct((N, 8), jnp.float32),
            jax.ShapeDtypeStruct((N, 1), jnp.float32),
            jax.ShapeDtypeStruct((G, 2), jnp.float32),
            jax.ShapeDtypeStruct((G, 1), jnp.float32),
            jax.ShapeDtypeStruct((G, 32), jnp.float32),
            jax.ShapeDtypeStruct((G, 32), jnp.float32),
        ],
    )(x, accs[0], accs[1], cnt0, cnt1, batch2,
      ne_w2a[:, 0:4].T, ne_w2a[:, 4:36].T, r1(ne_b2a),
      ne_w2b.T, r1(ne_b2b),
      ge_w1.T, r1(ge_b1), ge_w2.T, r1(ge_b2),
      ed_w1[:, 4:36].T)

    gdr, gdc = _sc_gather(xd, row3, col3)
    dr_p = gdr.reshape(E * 8 // 128, 128)
    dc_p = gdc.reshape(E * 8 // 128, 128)

    M2 = (nd_w1a[:, 2:34] @ ed_w2).T
    c2 = nd_b1a + ed_b2 @ nd_w1a[:, 2:34].T
    D_a = z832.at[0:2].set(ed_w1[:, 0:2].T)
    D_b = z832.at[0:2].set(ed_w1[:, 2:4].T)
    D_ndx = z832.at[0:2].set(nd_w1a[:, 0:2].T)
    B16a = kron(I16, D_a)
    B16b = kron(I16, D_b)
    B4m2 = kron(I4, M2)
    B16ndx = kron(I16, D_ndx)
    B4ndb = kron(I4, nd_w1b.T)
    bd14 = jnp.tile(ed_b1, 4).reshape(1, 128)
    c24 = jnp.tile(c2, 4).reshape(1, 128)
    bndb4 = jnp.tile(nd_b1b, 4).reshape(1, 128)
    S1 = jnp.zeros((8, 64), jnp.float32).at[2, :].set(1.0)
    Sg16 = kron(I16, S1)
    i64 = jnp.tile(jnp.arange(64, dtype=jnp.float32), 4).reshape(1, 256)
    umask = kron(I4, jnp.ones((64, 32), jnp.float32))

    hd128 = pl.pallas_call(
        _dec_edge_body,
        grid=(E // EB,),
        in_specs=[
            _blk((PB, 128)), _blk((PB, 128)), _full((G, 32)),
            _full((128, 512)), _full((128, 512)),
            _full((128, 1024)), _full((1, 256)), _full((256, 128)),
            _full((1, 128)), _full((128, 128)), _full((128, 512)),
            _full((1, 128)), _full((128, 128)), _full((1, 128)),
        ],
        out_specs=_blk((EB // 4, 128)),
        out_shape=jax.ShapeDtypeStruct((E * 32 // 128, 128), jnp.float32),
    )(dr_p, dc_p, urow, B16a, B16b, Sg16, i64, umask, bd14, B4m2, B16ndx,
      c24, B4ndb, bndb4)

    daccs = _scatter32(hd128.reshape(Q, SUB, CH, 32), col3, zer32)

    out = pl.pallas_call(
        _node_dec_body,
        grid=(N // NB,),
        in_specs=[
            _blk((NB, 2)), _blk((NB, 32)), _blk((NB, 32)), _blk((NB, 1)),
            _blk((NB, 1)), _full((G, 32)),
            _full((2, 32)), _full((32, 32)), _full((32, 32)), _full((1, 32)),
            _full((32, 4)), _full((1, 4)),
        ],
        out_specs=_blk((NB, 4)),
        out_shape=jax.ShapeDtypeStruct((N, 4), jnp.float32),
    )(x1, daccs[0], daccs[1], cntN, batch2, u,
      nd_w2a[:, 0:2].T, nd_w2a[:, 2:34].T, nd_w2a[:, 34:66].T, r1(nd_b2a),
      nd_w2b.T, r1(nd_b2b))

    return out
